# baseline (speedup 1.0000x reference)
import functools

import jax
import jax.numpy as jnp
from jax import lax
from jax.experimental import pallas as pl
from jax.experimental.pallas import tpu as pltpu
from jax.experimental.pallas import tpu_sc as plsc

N_NODES = 10000
D = 128

NC = 2
NS = 16
NW = NC * NS

CHUNK = 64
RB = 160
EPW = RB * CHUNK
E_PAD = NW * EPW
NPAD = 10112
ROWS_PER_TILE = NPAD // NS
NDUMMY = 96
GN = 4
IN_ = 16
IPRIME = 14
LOOK = 2
UNROLL = 16
HSZ = NPAD


def _enc_body(x_ref, w_ref, b_ref, o_ref):
    o_ref[...] = (
        jnp.dot(x_ref[...], w_ref[...], preferred_element_type=jnp.float32)
        + b_ref[...]
    )


def _encoder(x, w_t, b2):
    blk = 1000
    return pl.pallas_call(
        _enc_body,
        grid=(N_NODES // blk,),
        in_specs=[
            pl.BlockSpec((blk, D), lambda i: (i, 0)),
            pl.BlockSpec((D, D), lambda i: (0, 0)),
            pl.BlockSpec((1, D), lambda i: (0, 0)),
        ],
        out_specs=pl.BlockSpec((blk, D), lambda i: (i, 0)),
        out_shape=jax.ShapeDtypeStruct((N_NODES, D), jnp.float32),
    )(x, w_t, b2)


def _sc_body(h_hbm, idx_hbm, msg_out, cnt_out,
             ibuf, gbuf, hist,
             g0, g1, g2, g3, s0, s1, s2, s3,
             i0, i1, i2, i3, i4, i5, i6, i7,
             i8, i9, i10, i11, i12, i13, i14, i15,
             msg_sh):
    gsems = (g0, g1, g2, g3)
    ssems = (s0, s1, s2, s3)
    isems = (i0, i1, i2, i3, i4, i5, i6, i7,
             i8, i9, i10, i11, i12, i13, i14, i15)
    c = lax.axis_index("c")
    s = lax.axis_index("s")
    wid = c * NS + s

    z16 = jnp.zeros((16,), jnp.float32)

    def zero_hist(i, carry):
        hist[pl.ds(i * 16, 16)] = z16
        return carry

    lax.fori_loop(0, HSZ // 16, zero_hist, 0)

    for i in range(16):
        for j in range(D // 16):
            gbuf[0, i, pl.ds(j * 16, 16)] = z16
    zt = gbuf.at[0, pl.ds(0, 16)]
    base = s * ROWS_PER_TILE
    for k in range(ROWS_PER_TILE // 16):
        pltpu.sync_copy(zt, msg_sh.at[pl.ds(base + k * 16, 16)])
    rem = ROWS_PER_TILE % 16
    if rem:
        pltpu.sync_copy(
            gbuf.at[0, pl.ds(0, rem)],
            msg_sh.at[pl.ds(base + ROWS_PER_TILE - rem, rem)],
        )

    for t in range(IPRIME):
        pltpu.make_async_copy(idx_hbm.at[wid, t], ibuf.at[t], isems[t]).start()
    for q in range(LOOK):
        pltpu.make_async_copy(idx_hbm.at[wid, q], ibuf.at[q], isems[q]).wait()
        pltpu.make_async_copy(
            h_hbm.at[ibuf.at[q, 0]], gbuf.at[q], gsems[q]
        ).start()
    plsc.subcore_barrier()

    one16 = jnp.full((16,), 1.0, jnp.float32)
    n_outer = RB // UNROLL

    def outer(gg, carry):
        for b in range(UNROLL):
            g = gg * UNROLL + b
            q = b % GN
            pltpu.make_async_copy(
                h_hbm.at[ibuf.at[b, 0]], gbuf.at[q], gsems[q]
            ).wait()
            pltpu.async_copy(
                gbuf.at[q], msg_sh.at[ibuf.at[b, 1]], ssems[q], add=True
            )
            for k in range(CHUNK // 16):
                dv = ibuf[b, 1, pl.ds(k * 16, 16)]
                plsc.addupdate_scatter(hist, [dv], one16)

            @pl.when(g + LOOK < RB)
            def _():
                q2 = (b + LOOK) % GN
                nslot = (b + LOOK) % IN_

                @pl.when(g + LOOK >= GN)
                def _():
                    pltpu.make_async_copy(
                        gbuf.at[q2],
                        msg_sh.at[ibuf.at[(b + LOOK - GN) % IN_, 1]],
                        ssems[q2],
                    ).wait()

                pltpu.make_async_copy(
                    idx_hbm.at[wid, g + LOOK], ibuf.at[nslot], isems[nslot]
                ).wait()
                pltpu.make_async_copy(
                    h_hbm.at[ibuf.at[nslot, 0]], gbuf.at[q2], gsems[q2]
                ).start()

            @pl.when(g + IPRIME < RB)
            def _():
                pltpu.make_async_copy(
                    idx_hbm.at[wid, g + IPRIME],
                    ibuf.at[(b + IPRIME) % IN_],
                    isems[(b + IPRIME) % IN_],
                ).start()
        return carry

    lax.fori_loop(0, n_outer, outer, 0)
    for q in range(GN):
        g_last = RB - GN + q
        pltpu.make_async_copy(
            gbuf.at[(g_last % GN)], msg_sh.at[ibuf.at[g_last % IN_, 1]],
            ssems[g_last % GN],
        ).wait()
    plsc.subcore_barrier()

    pltpu.sync_copy(
        msg_sh.at[pl.ds(base, ROWS_PER_TILE)],
        msg_out.at[c, pl.ds(base, ROWS_PER_TILE)],
    )
    pltpu.sync_copy(hist, cnt_out.at[wid])


_sc_scatter = functools.partial(
    pl.kernel,
    out_type=(
        jax.ShapeDtypeStruct((NC, NPAD, D), jnp.float32),
        jax.ShapeDtypeStruct((NW, HSZ), jnp.float32),
    ),
    mesh=plsc.VectorSubcoreMesh(core_axis_name="c", subcore_axis_name="s"),
    compiler_params=pltpu.CompilerParams(needs_layout_passes=False),
    scratch_types=[
        pltpu.VMEM((IN_, 2, CHUNK), jnp.int32),
        pltpu.VMEM((GN, CHUNK, D), jnp.float32),
        pltpu.VMEM((HSZ,), jnp.float32),
    ] + [pltpu.SemaphoreType.DMA] * (2 * GN + IN_) + [
        pltpu.VMEM_SHARED((NPAD, D), jnp.float32),
    ],
)(_sc_body)


def _comb_body(h_ref, m0, m1, c_ref, wa, ba, o_ref):
    h = h_ref[...]
    msg = m0[...][0] + m1[...][0]
    cnt = jnp.sum(c_ref[...], axis=0)[:, None]
    neigh = msg / jnp.maximum(cnt, 1.0)
    h2 = h + jnp.where(cnt > 0.0, neigh, h)
    z = jnp.sum(h2 * wa[...], axis=1, keepdims=True)
    attn = jax.nn.sigmoid(z + ba[0, 0])
    o_ref[...] = h2 * attn


def _combine(h, msg_p, cnt_p, w_attn, b_attn2):
    blk = 1280
    return pl.pallas_call(
        _comb_body,
        grid=((N_NODES + blk - 1) // blk,),
        in_specs=[
            pl.BlockSpec((blk, D), lambda i: (i, 0)),
            pl.BlockSpec((1, blk, D), lambda i: (0, i, 0)),
            pl.BlockSpec((1, blk, D), lambda i: (1, i, 0)),
            pl.BlockSpec((NW, blk), lambda i: (0, i)),
            pl.BlockSpec((1, D), lambda i: (0, 0)),
            pl.BlockSpec((1, 1), lambda i: (0, 0)),
        ],
        out_specs=pl.BlockSpec((blk, D), lambda i: (i, 0)),
        out_shape=jax.ShapeDtypeStruct((N_NODES, D), jnp.float32),
    )(h, msg_p, msg_p, cnt_p, w_attn, b_attn2)


def kernel(x, edge_index, W_enc, b_enc, W_attn, b_attn):
    src = edge_index[0].astype(jnp.int32)
    dst = edge_index[1].astype(jnp.int32)
    n_edges = src.shape[0]
    pad = E_PAD - n_edges
    ar = jnp.arange(pad, dtype=jnp.int32)
    src_p = jnp.concatenate([src, (ar * 131) % N_NODES])
    dst_p = jnp.concatenate([dst, N_NODES + (ar % NDUMMY)])
    idx = jnp.stack(
        [src_p.reshape(NW, RB, CHUNK), dst_p.reshape(NW, RB, CHUNK)], axis=2
    )

    h = _encoder(x, W_enc.T, b_enc.reshape(1, D))
    msg_p, cnt_p = _sc_scatter(h, idx)
    return _combine(h, msg_p, cnt_p, W_attn, b_attn.reshape(1, 1))


# speedup vs baseline: 1.4005x; 1.4005x over previous
import functools

import jax
import jax.numpy as jnp
from jax import lax
from jax.experimental import pallas as pl
from jax.experimental.pallas import tpu as pltpu
from jax.experimental.pallas import tpu_sc as plsc

N_NODES = 10000
D = 128

NC = 2
NS = 16
NW = NC * NS

CHUNK = 64
RB = 160
EPW = RB * CHUNK
E_PAD = NW * EPW
NPAD = 10112
ROWS_PER_TILE = NPAD // NS
NDUMMY = 96
GN = 4
IN_ = 8
UNROLL = 8
HSZ = NPAD


def _enc_body(x_ref, w_ref, b_ref, o_ref):
    o_ref[...] = (
        jnp.dot(x_ref[...], w_ref[...], preferred_element_type=jnp.float32)
        + b_ref[...]
    )


def _encoder(x, w_t, b2):
    blk = 1000
    return pl.pallas_call(
        _enc_body,
        grid=(N_NODES // blk,),
        in_specs=[
            pl.BlockSpec((blk, D), lambda i: (i, 0)),
            pl.BlockSpec((D, D), lambda i: (0, 0)),
            pl.BlockSpec((1, D), lambda i: (0, 0)),
        ],
        out_specs=pl.BlockSpec((blk, D), lambda i: (i, 0)),
        out_shape=jax.ShapeDtypeStruct((N_NODES, D), jnp.float32),
    )(x, w_t, b2)


def _sc_body(h_hbm, edge_hbm, tail_hbm, msg_out, cnt_out,
             ibuf, gbuf, hist,
             g0, g1, g2, g3, i0, i1, i2, i3, i4, i5, i6, i7,
             msg_sh):
    gsems = (g0, g1, g2, g3)
    isems = (i0, i1, i2, i3, i4, i5, i6, i7)
    c = lax.axis_index("c")
    s = lax.axis_index("s")
    wid = c * NS + s
    ebase = wid * EPW

    def stage_idx(t_chunk, slot):
        @pl.when(wid < NW - 1)
        def _():
            off = ebase + t_chunk * CHUNK
            pltpu.make_async_copy(
                edge_hbm.at[0, pl.ds(off, CHUNK)], ibuf.at[slot, 0],
                isems[slot]).start()
            pltpu.make_async_copy(
                edge_hbm.at[1, pl.ds(off, CHUNK)], ibuf.at[slot, 1],
                isems[slot]).start()

        @pl.when(wid == NW - 1)
        def _():
            off = t_chunk * CHUNK
            pltpu.make_async_copy(
                tail_hbm.at[0, pl.ds(off, CHUNK)], ibuf.at[slot, 0],
                isems[slot]).start()
            pltpu.make_async_copy(
                tail_hbm.at[1, pl.ds(off, CHUNK)], ibuf.at[slot, 1],
                isems[slot]).start()

    def wait_idx(slot):
        for half in range(2):
            pltpu.make_async_copy(
                edge_hbm.at[0, pl.ds(0, CHUNK)], ibuf.at[slot, half],
                isems[slot]).wait()

    z16 = jnp.zeros((16,), jnp.float32)

    def zero_hist(i, carry):
        hist[pl.ds(i * 16, 16)] = z16
        return carry

    lax.fori_loop(0, HSZ // 16, zero_hist, 0)

    for t in range(IN_):
        stage_idx(t, t)

    for i in range(16):
        for j in range(D // 16):
            gbuf[0, i, pl.ds(j * 16, 16)] = z16
    zt = gbuf.at[0, pl.ds(0, 16)]
    base = s * ROWS_PER_TILE
    nz = ROWS_PER_TILE // 16
    for k0 in range(0, nz, 8):
        kk = min(8, nz - k0)
        for k in range(k0, k0 + kk):
            pltpu.make_async_copy(
                zt, msg_sh.at[pl.ds(base + k * 16, 16)], gsems[0]).start()
        for k in range(k0, k0 + kk):
            pltpu.make_async_copy(
                zt, msg_sh.at[pl.ds(base + k * 16, 16)], gsems[0]).wait()
    rem = ROWS_PER_TILE % 16
    if rem:
        pltpu.sync_copy(
            gbuf.at[0, pl.ds(0, rem)],
            msg_sh.at[pl.ds(base + ROWS_PER_TILE - rem, rem)],
        )

    for q in range(GN):
        wait_idx(q)
        pltpu.make_async_copy(
            h_hbm.at[ibuf.at[q, 0]], gbuf.at[q], gsems[q]
        ).start()
    plsc.subcore_barrier()

    one16 = jnp.full((16,), 1.0, jnp.float32)
    n_outer = RB // UNROLL

    def outer(gg, carry):
        for b in range(UNROLL):
            g = gg * UNROLL + b
            q = b % GN
            pltpu.make_async_copy(
                h_hbm.at[ibuf.at[b, 0]], gbuf.at[q], gsems[q]
            ).wait()
            pltpu.sync_copy(gbuf.at[q], msg_sh.at[ibuf.at[b, 1]], add=True)
            for k in range(CHUNK // 16):
                dv = ibuf[b, 1, pl.ds(k * 16, 16)]
                plsc.addupdate_scatter(hist, [dv], one16)

            @pl.when(g + IN_ < RB)
            def _():
                stage_idx(g + IN_, b)

            @pl.when(g + GN < RB)
            def _():
                nslot = (b + GN) % IN_
                wait_idx(nslot)
                pltpu.make_async_copy(
                    h_hbm.at[ibuf.at[nslot, 0]], gbuf.at[q], gsems[q]
                ).start()
        return carry

    lax.fori_loop(0, n_outer, outer, 0)
    plsc.subcore_barrier()

    pltpu.sync_copy(
        msg_sh.at[pl.ds(base, ROWS_PER_TILE)],
        msg_out.at[c, pl.ds(base, ROWS_PER_TILE)],
    )
    pltpu.sync_copy(hist, cnt_out.at[wid])


_sc_scatter = functools.partial(
    pl.kernel,
    out_type=(
        jax.ShapeDtypeStruct((NC, NPAD, D), jnp.float32),
        jax.ShapeDtypeStruct((NW, HSZ), jnp.float32),
    ),
    mesh=plsc.VectorSubcoreMesh(core_axis_name="c", subcore_axis_name="s"),
    compiler_params=pltpu.CompilerParams(needs_layout_passes=False),
    scratch_types=[
        pltpu.VMEM((IN_, 2, CHUNK), jnp.int32),
        pltpu.VMEM((GN, CHUNK, D), jnp.float32),
        pltpu.VMEM((HSZ,), jnp.float32),
    ] + [pltpu.SemaphoreType.DMA] * (GN + IN_) + [
        pltpu.VMEM_SHARED((NPAD, D), jnp.float32),
    ],
)(_sc_body)


def _comb_body(h_ref, m0, m1, c_ref, wa, ba, o_ref):
    h = h_ref[...]
    msg = m0[...][0] + m1[...][0]
    cnt = jnp.sum(c_ref[...], axis=0)[:, None]
    neigh = msg / jnp.maximum(cnt, 1.0)
    h2 = h + jnp.where(cnt > 0.0, neigh, h)
    z = jnp.sum(h2 * wa[...], axis=1, keepdims=True)
    attn = jax.nn.sigmoid(z + ba[0, 0])
    o_ref[...] = h2 * attn


def _combine(h, msg_p, cnt_p, w_attn, b_attn2):
    blk = 1280
    return pl.pallas_call(
        _comb_body,
        grid=((N_NODES + blk - 1) // blk,),
        in_specs=[
            pl.BlockSpec((blk, D), lambda i: (i, 0)),
            pl.BlockSpec((1, blk, D), lambda i: (0, i, 0)),
            pl.BlockSpec((1, blk, D), lambda i: (1, i, 0)),
            pl.BlockSpec((NW, blk), lambda i: (0, i)),
            pl.BlockSpec((1, D), lambda i: (0, 0)),
            pl.BlockSpec((1, 1), lambda i: (0, 0)),
        ],
        out_specs=pl.BlockSpec((blk, D), lambda i: (i, 0)),
        out_shape=jax.ShapeDtypeStruct((N_NODES, D), jnp.float32),
    )(h, msg_p, msg_p, cnt_p, w_attn, b_attn2)


def kernel(x, edge_index, W_enc, b_enc, W_attn, b_attn):
    e32 = edge_index.astype(jnp.int32)
    n_edges = e32.shape[1]
    tail_real = n_edges - (NW - 1) * EPW
    pad = EPW - tail_real
    ar = jnp.arange(pad, dtype=jnp.int32)
    tail = jnp.concatenate(
        [e32[:, (NW - 1) * EPW:],
         jnp.stack([(ar * 131) % N_NODES, N_NODES + (ar % NDUMMY)])],
        axis=1,
    )

    h = _encoder(x, W_enc.T, b_enc.reshape(1, D))
    msg_p, cnt_p = _sc_scatter(h, e32, tail)
    return _combine(h, msg_p, cnt_p, W_attn, b_attn.reshape(1, 1))
